# baseline (speedup 1.0000x reference)
import functools

import numpy as np
import jax
import jax.numpy as jnp
from jax import lax
from jax.experimental import pallas as pl
from jax.experimental.pallas import tpu as pltpu
from jax.experimental.pallas import tpu_sc as plsc

B, L, V, D, OF, C = 16384, 50, 1000000, 64, 128, 100
NC, NS = 2, 16
NW = NC * NS
BAGS_W = B // NW
IDX_W = BAGS_W * L
CHUNK = 128
SDIM = 1
CROWS = SDIM * CHUNK
NCHUNK = IDX_W // CROWS

_DST_NP = (np.arange(IDX_W, dtype=np.int32) // L)[None, :] + \
    (np.arange(NS, dtype=np.int32) * BAGS_W)[:, None]
_DST_TABLE = _DST_NP.reshape(NS, NCHUNK, CHUNK)


def _sc_bagsum_body(text_r, emb_r, dstt_r, zeros_r, out_r,
                    idx_v, dst_v, rows_v, acc_sh, gsem, ssem):
    c = lax.axis_index("c")
    s = lax.axis_index("s")
    wid = c * NS + s

    pltpu.sync_copy(text_r.at[wid], idx_v)
    pltpu.sync_copy(dstt_r.at[s], dst_v)
    pltpu.sync_copy(zeros_r, acc_sh.at[pl.ds(s * BAGS_W, BAGS_W)])

    pltpu.async_copy(emb_r.at[idx_v.at[0]], rows_v.at[0], gsem)
    pltpu.make_async_copy(emb_r.at[idx_v.at[0]], rows_v.at[0], gsem).wait()
    pltpu.async_copy(emb_r.at[idx_v.at[1]], rows_v.at[1], gsem)
    pltpu.async_copy(rows_v.at[0], acc_sh.at[dst_v.at[0]], ssem, add=True)

    def body(k, carry):
        p = lax.rem(k, 2)
        pltpu.make_async_copy(emb_r.at[idx_v.at[k]], rows_v.at[p],
                              gsem).wait()
        pltpu.make_async_copy(rows_v.at[1 - p],
                              acc_sh.at[dst_v.at[k - 1]], ssem).wait()
        pltpu.async_copy(emb_r.at[idx_v.at[k + 1]], rows_v.at[1 - p], gsem)
        pltpu.async_copy(rows_v.at[p], acc_sh.at[dst_v.at[k]], ssem,
                         add=True)
        return carry

    lax.fori_loop(1, NCHUNK - 1, body, 0)

    last = NCHUNK - 1
    p_last = last % 2
    pltpu.make_async_copy(emb_r.at[idx_v.at[last]], rows_v.at[p_last],
                          gsem).wait()
    pltpu.make_async_copy(rows_v.at[1 - p_last],
                          acc_sh.at[dst_v.at[last - 1]], ssem).wait()
    pltpu.async_copy(rows_v.at[p_last], acc_sh.at[dst_v.at[last]], ssem,
                     add=True)
    pltpu.make_async_copy(rows_v.at[p_last], acc_sh.at[dst_v.at[last]],
                          ssem).wait()

    pltpu.sync_copy(acc_sh.at[pl.ds(s * BAGS_W, BAGS_W)],
                    out_r.at[pl.ds(wid * BAGS_W, BAGS_W)])


def _bag_sums_sc(text_w, emb_table, dst_table, zeros):
    mesh = plsc.VectorSubcoreMesh(core_axis_name="c", subcore_axis_name="s")
    return pl.kernel(
        _sc_bagsum_body,
        out_type=jax.ShapeDtypeStruct((B, D), jnp.float32),
        mesh=mesh,
        scratch_types=[
            pltpu.VMEM((NCHUNK, CHUNK), jnp.int32),
            pltpu.VMEM((NCHUNK, CHUNK), jnp.int32),
            pltpu.VMEM((2, CROWS, D), jnp.float32),
            pltpu.VMEM_SHARED((NS * BAGS_W, D), jnp.float32),
            pltpu.SemaphoreType.DMA,
            pltpu.SemaphoreType.DMA,
        ],
        compiler_params=pltpu.CompilerParams(use_tc_tiling_on_sc=False),
    )(text_w, emb_table, dst_table, zeros)


VP = V // 2
RB = 16384
NRB = (VP + RB - 1) // RB
VP_PAD = NRB * RB
NCOL_LAST = (V + RB - 1) // RB - 1


def _tc_transpose_body(lo_r, hi_r, dst_r):
    dst_r[:, :D] = lo_r[...].T
    dst_r[:, D:] = hi_r[...].T


def _tc_transpose(emb_t):
    return pl.pallas_call(
        _tc_transpose_body,
        grid=(NRB,),
        in_specs=[
            pl.BlockSpec((D, RB), lambda i: (0, 2 * i)),
            pl.BlockSpec(
                (D, RB),
                lambda i: (0, jnp.minimum(2 * i + 1, NCOL_LAST))),
        ],
        out_specs=pl.BlockSpec((RB, 2 * D), lambda i: (i, 0)),
        out_shape=jax.ShapeDtypeStruct((VP_PAD, 2 * D), jnp.float32),
    )(emb_t, emb_t)


def _phys_row(r):
    return 2 * ((r // (2 * RB)) * RB + (r % RB)) + ((r // RB) % 2)


BM = 1024


def _tc_h_body(bag_r, text_r, other_r, w1_r, b1_r, wo_r, bo_r,
               h_r, sums_r, sumsq_r):
    i = pl.program_id(0)
    cnt = jnp.sum((text_r[...] != 0).astype(jnp.float32), axis=1,
                  keepdims=True)
    bag = bag_r[...] / jnp.maximum(cnt, 1.0)
    out1 = jnp.dot(bag, w1_r[...], preferred_element_type=jnp.float32) \
        + b1_r[...]
    out2 = jnp.dot(other_r[...], wo_r[...],
                   preferred_element_type=jnp.float32) + bo_r[...]
    h = jnp.maximum(jnp.concatenate([out1, out2], axis=1), 0.0)
    h_r[...] = h
    ps = jnp.sum(h, axis=0, keepdims=True)
    pq = jnp.sum(h * h, axis=0, keepdims=True)

    @pl.when(i == 0)
    def _():
        sums_r[...] = ps
        sumsq_r[...] = pq

    @pl.when(i > 0)
    def _():
        sums_r[...] += ps
        sumsq_r[...] += pq


def _tc_h(bag_sums, text_inputs, other_inputs, w1, b1, wo, bo):
    grid = (B // BM,)
    return pl.pallas_call(
        _tc_h_body,
        grid=grid,
        in_specs=[
            pl.BlockSpec((BM, D), lambda i: (i, 0)),
            pl.BlockSpec((BM, L), lambda i: (i, 0)),
            pl.BlockSpec((BM, OF), lambda i: (i, 0)),
            pl.BlockSpec((D, 128), lambda i: (0, 0)),
            pl.BlockSpec((1, 128), lambda i: (0, 0)),
            pl.BlockSpec((OF, 128), lambda i: (0, 0)),
            pl.BlockSpec((1, 128), lambda i: (0, 0)),
        ],
        out_specs=[
            pl.BlockSpec((BM, 256), lambda i: (i, 0)),
            pl.BlockSpec((1, 256), lambda i: (0, 0)),
            pl.BlockSpec((1, 256), lambda i: (0, 0)),
        ],
        out_shape=[
            jax.ShapeDtypeStruct((B, 256), jnp.float32),
            jax.ShapeDtypeStruct((1, 256), jnp.float32),
            jax.ShapeDtypeStruct((1, 256), jnp.float32),
        ],
    )(bag_sums, text_inputs, other_inputs, w1, b1, wo, bo)


def _tc_out_body(h_r, sums_r, sumsq_r, gamma_r, beta_r, w2_r, b2_r, out_r):
    mu = sums_r[...] * (1.0 / B)
    var = sumsq_r[...] * (1.0 / B) - mu * mu
    scale = gamma_r[...] / jnp.sqrt(var + 1e-5)
    hn = (h_r[...] - mu) * scale + beta_r[...]
    out_r[...] = jnp.dot(hn, w2_r[...],
                         preferred_element_type=jnp.float32) + b2_r[...]


def _tc_out(h, sums, sumsq, gamma, beta, w2, b2):
    grid = (B // BM,)
    return pl.pallas_call(
        _tc_out_body,
        grid=grid,
        in_specs=[
            pl.BlockSpec((BM, 256), lambda i: (i, 0)),
            pl.BlockSpec((1, 256), lambda i: (0, 0)),
            pl.BlockSpec((1, 256), lambda i: (0, 0)),
            pl.BlockSpec((1, 256), lambda i: (0, 0)),
            pl.BlockSpec((1, 256), lambda i: (0, 0)),
            pl.BlockSpec((256, C), lambda i: (0, 0)),
            pl.BlockSpec((1, C), lambda i: (0, 0)),
        ],
        out_specs=pl.BlockSpec((BM, C), lambda i: (i, 0)),
        out_shape=jax.ShapeDtypeStruct((B, C), jnp.float32),
    )(h, sums, sumsq, gamma, beta, w2, b2)


def kernel(text_inputs, other_inputs, emb_table, W1, b1, Wo, bo,
           gamma, beta, W2, b2):
    text_w = _phys_row(text_inputs).reshape(NW, NCHUNK, CHUNK)
    dst_table = jnp.asarray(_DST_TABLE)
    zeros = jnp.zeros((BAGS_W, D), jnp.float32)
    emb_packed = _tc_transpose(emb_table.T)
    bag_sums = _bag_sums_sc(text_w, emb_packed.reshape(2 * VP_PAD, D),
                            dst_table, zeros)
    h, sums, sumsq = _tc_h(
        bag_sums, text_inputs, other_inputs, W1,
        b1.reshape(1, 128), Wo, bo.reshape(1, 128))
    out = _tc_out(h, sums, sumsq, gamma.reshape(1, 256), beta.reshape(1, 256),
                  W2, b2.reshape(1, C))
    return out


# speedup vs baseline: 1.0017x; 1.0017x over previous
import functools

import numpy as np
import jax
import jax.numpy as jnp
from jax import lax
from jax.experimental import pallas as pl
from jax.experimental.pallas import tpu as pltpu
from jax.experimental.pallas import tpu_sc as plsc

B, L, V, D, OF, C = 16384, 50, 1000000, 64, 128, 100
NC, NS = 2, 16
NW = NC * NS
BAGS_W = B // NW
IDX_W = BAGS_W * L
CHUNK = 128
SDIM = 1
CROWS = SDIM * CHUNK
NCHUNK = IDX_W // CROWS

_DST_NP = (np.arange(IDX_W, dtype=np.int32) // L)[None, :] + \
    (np.arange(NS, dtype=np.int32) * BAGS_W)[:, None]
_DST_TABLE = _DST_NP.reshape(NS, NCHUNK, CHUNK)


def _sc_bagsum_body(text_r, emb_r, dstt_r, zeros_r, out_r,
                    idx_v, dst_v, rows_v, acc_sh, gsem):
    c = lax.axis_index("c")
    s = lax.axis_index("s")
    wid = c * NS + s

    pltpu.sync_copy(text_r.at[wid], idx_v)
    pltpu.sync_copy(dstt_r.at[s], dst_v)
    pltpu.sync_copy(zeros_r, acc_sh.at[pl.ds(s * BAGS_W, BAGS_W)])

    pltpu.async_copy(emb_r.at[idx_v.at[0]], rows_v.at[0], gsem)

    def body(k, carry):
        p = lax.rem(k, 2)
        pltpu.make_async_copy(emb_r.at[idx_v.at[k]], rows_v.at[p],
                              gsem).wait()
        pltpu.async_copy(emb_r.at[idx_v.at[k + 1]], rows_v.at[1 - p], gsem)
        pltpu.sync_copy(rows_v.at[p], acc_sh.at[dst_v.at[k]], add=True)
        return carry

    lax.fori_loop(0, NCHUNK - 1, body, 0)

    last = NCHUNK - 1
    p_last = last % 2
    pltpu.make_async_copy(emb_r.at[idx_v.at[last]], rows_v.at[p_last],
                          gsem).wait()
    pltpu.sync_copy(rows_v.at[p_last], acc_sh.at[dst_v.at[last]], add=True)

    pltpu.sync_copy(acc_sh.at[pl.ds(s * BAGS_W, BAGS_W)],
                    out_r.at[pl.ds(wid * BAGS_W, BAGS_W)])


def _bag_sums_sc(text_w, emb_table, dst_table, zeros):
    mesh = plsc.VectorSubcoreMesh(core_axis_name="c", subcore_axis_name="s")
    return pl.kernel(
        _sc_bagsum_body,
        out_type=jax.ShapeDtypeStruct((B, D), jnp.float32),
        mesh=mesh,
        scratch_types=[
            pltpu.VMEM((NCHUNK, CHUNK), jnp.int32),
            pltpu.VMEM((NCHUNK, CHUNK), jnp.int32),
            pltpu.VMEM((2, CROWS, D), jnp.float32),
            pltpu.VMEM_SHARED((NS * BAGS_W, D), jnp.float32),
            pltpu.SemaphoreType.DMA,
        ],
        compiler_params=pltpu.CompilerParams(use_tc_tiling_on_sc=False),
    )(text_w, emb_table, dst_table, zeros)


VP = V // 2
RB = 16384
NRB = (VP + RB - 1) // RB
VP_PAD = NRB * RB
NCOL_LAST = (V + RB - 1) // RB - 1


def _tc_transpose_body(lo_r, hi_r, dst_r):
    dst_r[:, :D] = lo_r[...].T
    dst_r[:, D:] = hi_r[...].T


def _tc_transpose(emb_t):
    return pl.pallas_call(
        _tc_transpose_body,
        grid=(NRB,),
        in_specs=[
            pl.BlockSpec((D, RB), lambda i: (0, 2 * i)),
            pl.BlockSpec(
                (D, RB),
                lambda i: (0, jnp.minimum(2 * i + 1, NCOL_LAST))),
        ],
        out_specs=pl.BlockSpec((RB, 2 * D), lambda i: (i, 0)),
        out_shape=jax.ShapeDtypeStruct((VP_PAD, 2 * D), jnp.float32),
    )(emb_t, emb_t)


def _phys_row(r):
    return 2 * ((r // (2 * RB)) * RB + (r % RB)) + ((r // RB) % 2)


BM = 1024


def _tc_h_body(bag_r, text_r, other_r, w1_r, b1_r, wo_r, bo_r,
               h_r, sums_r, sumsq_r):
    i = pl.program_id(0)
    cnt = jnp.sum((text_r[...] != 0).astype(jnp.float32), axis=1,
                  keepdims=True)
    bag = bag_r[...] / jnp.maximum(cnt, 1.0)
    out1 = jnp.dot(bag, w1_r[...], preferred_element_type=jnp.float32) \
        + b1_r[...]
    out2 = jnp.dot(other_r[...], wo_r[...],
                   preferred_element_type=jnp.float32) + bo_r[...]
    h = jnp.maximum(jnp.concatenate([out1, out2], axis=1), 0.0)
    h_r[...] = h
    ps = jnp.sum(h, axis=0, keepdims=True)
    pq = jnp.sum(h * h, axis=0, keepdims=True)

    @pl.when(i == 0)
    def _():
        sums_r[...] = ps
        sumsq_r[...] = pq

    @pl.when(i > 0)
    def _():
        sums_r[...] += ps
        sumsq_r[...] += pq


def _tc_h(bag_sums, text_inputs, other_inputs, w1, b1, wo, bo):
    grid = (B // BM,)
    return pl.pallas_call(
        _tc_h_body,
        grid=grid,
        in_specs=[
            pl.BlockSpec((BM, D), lambda i: (i, 0)),
            pl.BlockSpec((BM, L), lambda i: (i, 0)),
            pl.BlockSpec((BM, OF), lambda i: (i, 0)),
            pl.BlockSpec((D, 128), lambda i: (0, 0)),
            pl.BlockSpec((1, 128), lambda i: (0, 0)),
            pl.BlockSpec((OF, 128), lambda i: (0, 0)),
            pl.BlockSpec((1, 128), lambda i: (0, 0)),
        ],
        out_specs=[
            pl.BlockSpec((BM, 256), lambda i: (i, 0)),
            pl.BlockSpec((1, 256), lambda i: (0, 0)),
            pl.BlockSpec((1, 256), lambda i: (0, 0)),
        ],
        out_shape=[
            jax.ShapeDtypeStruct((B, 256), jnp.float32),
            jax.ShapeDtypeStruct((1, 256), jnp.float32),
            jax.ShapeDtypeStruct((1, 256), jnp.float32),
        ],
    )(bag_sums, text_inputs, other_inputs, w1, b1, wo, bo)


def _tc_out_body(h_r, sums_r, sumsq_r, gamma_r, beta_r, w2_r, b2_r, out_r):
    mu = sums_r[...] * (1.0 / B)
    var = sumsq_r[...] * (1.0 / B) - mu * mu
    scale = gamma_r[...] / jnp.sqrt(var + 1e-5)
    hn = (h_r[...] - mu) * scale + beta_r[...]
    out_r[...] = jnp.dot(hn, w2_r[...],
                         preferred_element_type=jnp.float32) + b2_r[...]


def _tc_out(h, sums, sumsq, gamma, beta, w2, b2):
    grid = (B // BM,)
    return pl.pallas_call(
        _tc_out_body,
        grid=grid,
        in_specs=[
            pl.BlockSpec((BM, 256), lambda i: (i, 0)),
            pl.BlockSpec((1, 256), lambda i: (0, 0)),
            pl.BlockSpec((1, 256), lambda i: (0, 0)),
            pl.BlockSpec((1, 256), lambda i: (0, 0)),
            pl.BlockSpec((1, 256), lambda i: (0, 0)),
            pl.BlockSpec((256, C), lambda i: (0, 0)),
            pl.BlockSpec((1, C), lambda i: (0, 0)),
        ],
        out_specs=pl.BlockSpec((BM, C), lambda i: (i, 0)),
        out_shape=jax.ShapeDtypeStruct((B, C), jnp.float32),
    )(h, sums, sumsq, gamma, beta, w2, b2)


def kernel(text_inputs, other_inputs, emb_table, W1, b1, Wo, bo,
           gamma, beta, W2, b2):
    text_w = _phys_row(text_inputs).reshape(NW, NCHUNK, CHUNK)
    dst_table = jnp.asarray(_DST_TABLE)
    zeros = jnp.zeros((BAGS_W, D), jnp.float32)
    emb_packed = _tc_transpose(emb_table.T)
    bag_sums = _bag_sums_sc(text_w, emb_packed.reshape(2 * VP_PAD, D),
                            dst_table, zeros)
    h, sums, sumsq = _tc_h(
        bag_sums, text_inputs, other_inputs, W1,
        b1.reshape(1, 128), Wo, bo.reshape(1, 128))
    out = _tc_out(h, sums, sumsq, gamma.reshape(1, 256), beta.reshape(1, 256),
                  W2, b2.reshape(1, C))
    return out
